# Initial kernel scaffold; baseline (speedup 1.0000x reference)
import jax
import jax.numpy as jnp
from jax.experimental import pallas as pl


def kernel(pts, params):
    raise NotImplementedError("write your pallas kernel here")


# speedup vs baseline: 6.4201x; 6.4201x over previous
import functools

import jax
import jax.numpy as jnp
import numpy as np
from jax import lax
from jax.experimental import pallas as pl
from jax.experimental.pallas import tpu as pltpu
from jax.experimental.pallas import tpu_sc as plsc

B, N, N1, N2, CGEO = 16, 4096, 512, 128, 256
R1, R2, KFP, MAXN1, MAXN2 = 0.2, 0.4, 3, 32, 64
EPS = 1e-5


def _fps_body(npoint, px, py, pz, idx_ref, ox, oy, oz, dists):
    Bb, Nn = px.shape
    dists[...] = jnp.full((Bb, Nn), 1e10, jnp.float32)
    colN = lax.broadcasted_iota(jnp.int32, (Bb, Nn), 1)
    colP = lax.broadcasted_iota(jnp.int32, (Bb, npoint), 1)

    def body(i, far):
        oh = colN == far
        pxv, pyv, pzv = px[...], py[...], pz[...]
        cx = jnp.sum(jnp.where(oh, pxv, 0.0), axis=1, keepdims=True)
        cy = jnp.sum(jnp.where(oh, pyv, 0.0), axis=1, keepdims=True)
        cz = jnp.sum(jnp.where(oh, pzv, 0.0), axis=1, keepdims=True)
        sm = colP == i
        idx_ref[...] = jnp.where(sm, far, idx_ref[...])
        ox[...] = jnp.where(sm, cx, ox[...])
        oy[...] = jnp.where(sm, cy, oy[...])
        oz[...] = jnp.where(sm, cz, oz[...])
        dx = pxv - cx
        dy = pyv - cy
        dz = pzv - cz
        d = dx * dx + dy * dy + dz * dz
        nd = jnp.minimum(dists[...], d)
        dists[...] = nd
        return jnp.argmax(nd, axis=1).astype(jnp.int32)[:, None]

    lax.fori_loop(0, npoint, body, jnp.zeros((Bb, 1), jnp.int32))


def _fps(px, py, pz, npoint):
    Bb, Nn = px.shape
    out_shape = (
        jax.ShapeDtypeStruct((Bb, npoint), jnp.int32),
        jax.ShapeDtypeStruct((Bb, npoint), jnp.float32),
        jax.ShapeDtypeStruct((Bb, npoint), jnp.float32),
        jax.ShapeDtypeStruct((Bb, npoint), jnp.float32),
    )
    return pl.pallas_call(
        functools.partial(_fps_body, npoint),
        out_shape=out_shape,
        scratch_shapes=[pltpu.VMEM((Bb, Nn), jnp.float32)],
    )(px, py, pz)


def _ballq_body(r2, S, G, GS, NW, sx, sy, sz, qx, qy, qz, pmat, out_ref):
    Qb = qx.shape[2]
    dx = qx[0, 0, :][:, None] - sx[0, 0, :][None, :]
    dy = qy[0, 0, :][:, None] - sy[0, 0, :][None, :]
    dz = qz[0, 0, :][:, None] - sz[0, 0, :][None, :]
    d2 = dx * dx + dy * dy + dz * dz
    mask = (d2 <= r2).astype(jnp.bfloat16)
    words = jnp.dot(mask, pmat[...], preferred_element_type=jnp.float32)
    cnt = words[:, :G]
    D = cnt
    k = 1
    while k < G:
        D = D + jnp.concatenate(
            [jnp.zeros((Qb, k), jnp.float32), D[:, : G - k]], axis=1)
        k *= 2
    Dm1i = (D - cnt).astype(jnp.int32)
    Di = D.astype(jnp.int32)
    T = Di[:, G - 1:G]
    wi = [words[:, (1 + w) * G:(2 + w) * G].astype(jnp.int32)
          for w in range(NW)]
    iota_g = lax.broadcasted_iota(jnp.int32, (Qb, G), 1)
    iota_s = lax.broadcasted_iota(jnp.int32, (Qb, S), 1)

    def seat(s, carry):
        gsel, rsel, wsel = carry
        le = (Di <= s).astype(jnp.int32)
        g = jnp.sum(le, axis=1, keepdims=True)
        oh = iota_g == g
        cg = jnp.sum(jnp.where(oh, Dm1i, 0), axis=1, keepdims=True)
        r = s - cg
        ws = [jnp.sum(jnp.where(oh, w, 0), axis=1, keepdims=True) for w in wi]
        sm = iota_s == s
        gsel = jnp.where(sm, g, gsel)
        rsel = jnp.where(sm, r, rsel)
        wsel = [jnp.where(sm, w, ww) for w, ww in zip(ws, wsel)]
        return gsel, rsel, wsel

    z = jnp.zeros((Qb, S), jnp.int32)
    gsel, rsel, wsel = lax.fori_loop(
        0, S, seat, (z, z, [z for _ in range(NW)]))

    cum = jnp.zeros((Qb, S), jnp.int32)
    pos = jnp.zeros((Qb, S), jnp.int32)
    for w in range(NW):
        for p in range(16):
            bit = (wsel[w] >> p) & 1
            hit = (bit == 1) & (cum == rsel)
            pos = jnp.where(hit, w * 16 + p, pos)
            cum = cum + bit
    idx = gsel * GS + pos
    first = idx[:, 0:1]
    out_ref[0] = jnp.where(iota_s < T, idx, first)


def _ball_query(sx, sy, sz, qx, qy, qz, r, S, GS=64, Qb=128):
    Bb, Ns = sx.shape
    Q = qx.shape[1]
    G = Ns // GS
    NW = GS // 16
    pm = np.zeros((Ns, (1 + NW) * G), np.float32)
    for i in range(Ns):
        g = i // GS
        j = i % GS
        pm[i, g] = 1.0
        pm[i, (1 + j // 16) * G + g] = float(1 << (j % 16))
    pmat = jnp.asarray(pm, jnp.bfloat16)
    grid = (Bb, Q // Qb)
    src_spec = pl.BlockSpec((1, 1, Ns), lambda b, q: (b, 0, 0))
    q_spec = pl.BlockSpec((1, 1, Qb), lambda b, q: (b, 0, q))
    return pl.pallas_call(
        functools.partial(_ballq_body, r * r, S, G, GS, NW),
        grid=grid,
        in_specs=[src_spec] * 3 + [q_spec] * 3
        + [pl.BlockSpec((Ns, (1 + NW) * G), lambda b, q: (0, 0))],
        out_specs=pl.BlockSpec((1, Qb, S), lambda b, q: (b, q, 0)),
        out_shape=jax.ShapeDtypeStruct((Bb, Q, S), jnp.int32),
    )(sx.reshape(Bb, 1, Ns), sy.reshape(Bb, 1, Ns), sz.reshape(Bb, 1, Ns),
      qx.reshape(Bb, 1, Q), qy.reshape(Bb, 1, Q), qz.reshape(Bb, 1, Q), pmat)


def _bn_relu(z, m_ref, v_ref, g_ref, bt_ref):
    y = ((z - m_ref[0:1, :]) / jnp.sqrt(v_ref[0:1, :] + EPS) * g_ref[0:1, :]
         + bt_ref[0:1, :])
    return jnp.maximum(y, 0.0)


def _row_spec(c, br):
    return pl.BlockSpec((br, c), lambda i: (i, 0))


def _full2(a):
    return pl.BlockSpec(a.shape, lambda *_: (0, 0))


def _crow(c):
    return pl.BlockSpec((1, c), lambda *_: (0, 0))


def _start_body(x_ref, w_ref, b_ref, z_ref):
    z_ref[...] = jnp.dot(
        x_ref[...].astype(jnp.bfloat16), w_ref[...].astype(jnp.bfloat16),
        preferred_element_type=jnp.float32) + b_ref[0:1, :]


def _mlp_start(x, w, b, br=2048):
    R, ci = x.shape
    co = w.shape[1]
    return pl.pallas_call(
        _start_body,
        grid=(R // br,),
        in_specs=[_row_spec(ci, br), _full2(w), _crow(co)],
        out_specs=_row_spec(co, br),
        out_shape=jax.ShapeDtypeStruct((R, co), jnp.float32),
    )(x, w, b[None, :])


def _link_body(z_ref, m_ref, v_ref, g_ref, bt_ref, w_ref, b_ref, z2_ref):
    y = _bn_relu(z_ref[...], m_ref, v_ref, g_ref, bt_ref)
    z2_ref[...] = jnp.dot(
        y.astype(jnp.bfloat16), w_ref[...].astype(jnp.bfloat16),
        preferred_element_type=jnp.float32) + b_ref[0:1, :]


def _mlp_link(z, m, v, g, bt, w, b, br=2048):
    R, ci = z.shape
    co = w.shape[1]
    return pl.pallas_call(
        _link_body,
        grid=(R // br,),
        in_specs=[_row_spec(ci, br), _crow(ci), _crow(ci), _crow(ci),
                  _crow(ci), _full2(w), _crow(co)],
        out_specs=_row_spec(co, br),
        out_shape=jax.ShapeDtypeStruct((R, co), jnp.float32),
    )(z, m, v, g[None, :], bt[None, :], w, b[None, :])


def _finish_body(z_ref, m_ref, v_ref, g_ref, bt_ref, y_ref):
    y_ref[...] = _bn_relu(z_ref[...], m_ref, v_ref, g_ref, bt_ref)


def _mlp_finish(z, m, v, g, bt, br=2048):
    R, ci = z.shape
    return pl.pallas_call(
        _finish_body,
        grid=(R // br,),
        in_specs=[_row_spec(ci, br), _crow(ci), _crow(ci), _crow(ci),
                  _crow(ci)],
        out_specs=_row_spec(ci, br),
        out_shape=jax.ShapeDtypeStruct((R, ci), jnp.float32),
    )(z, m, v, g[None, :], bt[None, :])


def _finish_max_body(K, z_ref, m_ref, v_ref, g_ref, bt_ref, y_ref):
    y = _bn_relu(z_ref[...], m_ref, v_ref, g_ref, bt_ref)
    rb, c = z_ref.shape
    y_ref[...] = jnp.max(y.reshape(rb // K, K, c), axis=1)


def _mlp_finish_max(z, m, v, g, bt, K, qb=128):
    R, ci = z.shape
    br = qb * K
    return pl.pallas_call(
        functools.partial(_finish_max_body, K),
        grid=(R // br,),
        in_specs=[_row_spec(ci, br), _crow(ci), _crow(ci), _crow(ci),
                  _crow(ci)],
        out_specs=_row_spec(ci, qb),
        out_shape=jax.ShapeDtypeStruct((R // K, ci), jnp.float32),
    )(z, m, v, g[None, :], bt[None, :])


def _start_max_body(Nn, x_ref, w_ref, b_ref, z_ref):
    R, c = x_ref.shape
    mx = jnp.max(x_ref[...].reshape(R // Nn, Nn, c), axis=1)
    z_ref[...] = jnp.dot(
        mx.astype(jnp.bfloat16), w_ref[...].astype(jnp.bfloat16),
        preferred_element_type=jnp.float32) + b_ref[0:1, :]


def _mlp_start_max(x, Nn, w, b):
    R, ci = x.shape
    co = w.shape[1]
    return pl.pallas_call(
        functools.partial(_start_max_body, Nn),
        out_shape=jax.ShapeDtypeStruct((R // Nn, co), jnp.float32),
    )(x, w, b[None, :])


def _sa_l1_body(Sn, xw, plo, cx, cy, cz, w_ref, b_ref, rows_ref, z_ref):
    rows = rows_ref[...]
    cb = jnp.concatenate(
        [cx[0, 0, :][:, None], cy[0, 0, :][:, None], cz[0, 0, :][:, None]],
        axis=1)
    qb = cb.shape[0]
    cbr = jnp.broadcast_to(cb[:, None, :], (qb, Sn, 3)).reshape(qb * Sn, 3)
    feat = jnp.concatenate(
        [rows[:, :xw], rows[:, plo:plo + 3] - cbr], axis=1)
    z_ref[...] = jnp.dot(
        feat.astype(jnp.bfloat16), w_ref[...].astype(jnp.bfloat16),
        preferred_element_type=jnp.float32) + b_ref[0:1, :]


def _sa_l1(rows, cx, cy, cz, w, b, Q, Sn, xw, plo, qb=128):
    R, dpad = rows.shape
    c = w.shape[1]
    Bb = cx.shape[0]
    nq = Q // qb
    cspec = pl.BlockSpec((1, 1, qb), lambda b_, q: (b_, 0, q))
    return pl.pallas_call(
        functools.partial(_sa_l1_body, Sn, xw, plo),
        grid=(Bb, nq),
        in_specs=[cspec, cspec, cspec, _full2(w),
                  pl.BlockSpec((1, c), lambda b_, q: (0, 0)),
                  pl.BlockSpec((qb * Sn, dpad),
                               lambda b_, q: (b_ * nq + q, 0))],
        out_specs=pl.BlockSpec((qb * Sn, c), lambda b_, q: (b_ * nq + q, 0)),
        out_shape=jax.ShapeDtypeStruct((R, c), jnp.float32),
    )(cx.reshape(Bb, 1, Q), cy.reshape(Bb, 1, Q), cz.reshape(Bb, 1, Q),
      w, b[None, :], rows)


def _interp_body(k, sx, sy, sz, tx, ty, tz, x_ref, o_ref):
    Tb = tx.shape[2]
    Ns = sx.shape[2]
    dx = tx[0, 0, :][:, None] - sx[0, 0, :][None, :]
    dy = ty[0, 0, :][:, None] - sy[0, 0, :][None, :]
    dz = tz[0, 0, :][:, None] - sz[0, 0, :][None, :]
    d2 = dx * dx + dy * dy + dz * dz
    iota = lax.broadcasted_iota(jnp.int32, (Tb, Ns), 1)
    rw = jnp.zeros((Tb, Ns), jnp.float32)
    denom = jnp.zeros((Tb, 1), jnp.float32)
    for _ in range(k):
        mn = jnp.min(d2, axis=1, keepdims=True)
        am = jnp.argmin(d2, axis=1).astype(jnp.int32)[:, None]
        w = 1.0 / jnp.maximum(mn, 1e-16)
        oh = iota == am
        rw = rw + jnp.where(oh, w, 0.0)
        denom = denom + w
        d2 = jnp.where(oh, jnp.float32(1e30), d2)
    y = jnp.dot(rw, x_ref[0], preferred_element_type=jnp.float32)
    o_ref[0] = y / denom


def _knn_interp(sx, sy, sz, tx, ty, tz, xsrc, tb=256):
    Bb, Ns = sx.shape
    Nt = tx.shape[1]
    C = xsrc.shape[2]
    sspec = pl.BlockSpec((1, 1, Ns), lambda b_, t: (b_, 0, 0))
    tspec = pl.BlockSpec((1, 1, tb), lambda b_, t: (b_, 0, t))
    return pl.pallas_call(
        functools.partial(_interp_body, KFP),
        grid=(Bb, Nt // tb),
        in_specs=[sspec, sspec, sspec, tspec, tspec, tspec,
                  pl.BlockSpec((1, Ns, C), lambda b_, t: (b_, 0, 0))],
        out_specs=pl.BlockSpec((1, tb, C), lambda b_, t: (b_, t, 0)),
        out_shape=jax.ShapeDtypeStruct((Bb, Nt, C), jnp.float32),
    )(sx.reshape(Bb, 1, Ns), sy.reshape(Bb, 1, Ns), sz.reshape(Bb, 1, Ns),
      tx.reshape(Bb, 1, Nt), ty.reshape(Bb, 1, Nt), tz.reshape(Bb, 1, Nt),
      xsrc)


def _sc_gather(table, idx):
    V, D = table.shape
    M = idx.shape[0]
    info = plsc.get_sparse_core_info()
    nw = info.num_cores * info.num_subcores
    per_w = M // nw
    assert per_w * nw == M
    ch = 128
    nch = per_w // ch
    assert nch * ch == per_w
    mesh = plsc.VectorSubcoreMesh(core_axis_name="c", subcore_axis_name="s")

    @functools.partial(
        pl.kernel, mesh=mesh,
        out_type=jax.ShapeDtypeStruct((M, D), jnp.float32),
        compiler_params=pltpu.CompilerParams(use_tc_tiling_on_sc=False),
        scratch_types=[pltpu.VMEM((ch,), jnp.int32),
                       pltpu.VMEM((ch, D), jnp.float32),
                       pltpu.SemaphoreType.DMA])
    def k(table_hbm, idx_hbm, out_hbm, idx_v, rows_v, sem):
        wid = lax.axis_index("s") * info.num_cores + lax.axis_index("c")
        base = wid * per_w

        def chunk(c_, carry):
            off = base + c_ * ch
            pltpu.sync_copy(idx_hbm.at[pl.ds(off, ch)], idx_v)
            pltpu.async_copy(table_hbm.at[idx_v], rows_v, sem).wait()
            pltpu.sync_copy(rows_v, out_hbm.at[pl.ds(off, ch)])
            return carry

        lax.fori_loop(0, nch, chunk, 0)

    return k(table, idx)


def _bn_relu_x(x, g, b):
    xs = x.reshape(-1, x.shape[-1])
    mean = xs.mean(axis=0)
    var = xs.var(axis=0)
    y = (x - mean) / jnp.sqrt(var + EPS) * g + b
    return jnp.maximum(y, 0.0)


def _apply_mlp_x(x, layers):
    for (Wl, bb, gm, bt) in layers:
        x = _bn_relu_x(x @ Wl + bb, gm, bt)
    return x


def kernel(pts, params):
    ptsx, ptsy, ptsz = pts[:, :, 0], pts[:, :, 1], pts[:, :, 2]

    _, p1x, p1y, p1z = _fps(ptsx, ptsy, ptsz, N1)
    pos1 = jnp.stack([p1x, p1y, p1z], axis=-1)
    g1 = _ball_query(ptsx, ptsy, ptsz, p1x, p1y, p1z, R1, MAXN1)
    tab1 = jnp.concatenate(
        [pts.reshape(B * N, 3),
         jnp.zeros((B * N, 13), jnp.float32)], axis=1)
    offs = (jnp.arange(B, dtype=jnp.int32) * N)[:, None]
    gidx1 = (g1.reshape(B, -1) + offs).reshape(-1)
    rows1 = _sc_gather(tab1, gidx1)
    pj = rows1[:, :3].reshape(B, N1, MAXN1, 3)
    feat = jnp.concatenate([pj, pj - pos1[:, :, None, :]], axis=-1)
    h = _apply_mlp_x(feat, params['sa1_local'])
    x1 = _apply_mlp_x(jnp.max(h, axis=2), params['sa1_global'])

    _, p2x, p2y, p2z = _fps(p1x, p1y, p1z, N2)
    pos2 = jnp.stack([p2x, p2y, p2z], axis=-1)
    g2 = _ball_query(p1x, p1y, p1z, p2x, p2y, p2z, R2, MAXN2)
    tab2 = jnp.concatenate(
        [x1.reshape(B * N1, 256), pos1.reshape(B * N1, 3),
         jnp.zeros((B * N1, 13), jnp.float32)], axis=1)
    offs1 = (jnp.arange(B, dtype=jnp.int32) * N1)[:, None]
    gidx2 = (g2.reshape(B, -1) + offs1).reshape(-1)
    rows2 = _sc_gather(tab2, gidx2)
    xj = rows2[:, :256].reshape(B, N2, MAXN2, 256)
    pj2 = rows2[:, 256:259].reshape(B, N2, MAXN2, 3)
    feat2 = jnp.concatenate([xj, pj2 - pos2[:, :, None, :]], axis=-1)
    hh = _apply_mlp_x(feat2, params['sa2_local'])
    x2 = _apply_mlp_x(jnp.max(hh, axis=2), params['sa2_global'])

    gg = _apply_mlp_x(jnp.max(x2, axis=1), params['glob'])

    x1_up = _knn_interp(p2x, p2y, p2z, p1x, p1y, p1z, x2, tb=256)
    x1_fp = _apply_mlp_x(jnp.concatenate([x1_up, x1], axis=-1),
                         params['fp1'])

    x0_up = _knn_interp(p1x, p1y, p1z, ptsx, ptsy, ptsz, x1_fp, tb=256)
    F = _apply_mlp_x(jnp.concatenate([x0_up, pts], axis=-1), params['fp0'])
    return (F, gg)


# speedup vs baseline: 6.4370x; 1.0026x over previous
import functools

import jax
import jax.numpy as jnp
import numpy as np
from jax import lax
from jax.experimental import pallas as pl
from jax.experimental.pallas import tpu as pltpu
from jax.experimental.pallas import tpu_sc as plsc

B, N, N1, N2, CGEO = 16, 4096, 512, 128, 256
R1, R2, KFP, MAXN1, MAXN2 = 0.2, 0.4, 3, 32, 64
EPS = 1e-5


def _fps_body(npoint, px, py, pz, idx_ref, ox, oy, oz, dists):
    Bb, Nn = px.shape
    dists[...] = jnp.full((Bb, Nn), 1e10, jnp.float32)
    colN = lax.broadcasted_iota(jnp.int32, (Bb, Nn), 1)
    colP = lax.broadcasted_iota(jnp.int32, (Bb, npoint), 1)

    def body(i, far):
        oh = colN == far
        pxv, pyv, pzv = px[...], py[...], pz[...]
        cx = jnp.sum(jnp.where(oh, pxv, 0.0), axis=1, keepdims=True)
        cy = jnp.sum(jnp.where(oh, pyv, 0.0), axis=1, keepdims=True)
        cz = jnp.sum(jnp.where(oh, pzv, 0.0), axis=1, keepdims=True)
        sm = colP == i
        idx_ref[...] = jnp.where(sm, far, idx_ref[...])
        ox[...] = jnp.where(sm, cx, ox[...])
        oy[...] = jnp.where(sm, cy, oy[...])
        oz[...] = jnp.where(sm, cz, oz[...])
        dx = pxv - cx
        dy = pyv - cy
        dz = pzv - cz
        d = dx * dx + dy * dy + dz * dz
        nd = jnp.minimum(dists[...], d)
        dists[...] = nd
        return jnp.argmax(nd, axis=1).astype(jnp.int32)[:, None]

    lax.fori_loop(0, npoint, body, jnp.zeros((Bb, 1), jnp.int32))


def _fps(px, py, pz, npoint):
    Bb, Nn = px.shape
    out_shape = (
        jax.ShapeDtypeStruct((Bb, npoint), jnp.int32),
        jax.ShapeDtypeStruct((Bb, npoint), jnp.float32),
        jax.ShapeDtypeStruct((Bb, npoint), jnp.float32),
        jax.ShapeDtypeStruct((Bb, npoint), jnp.float32),
    )
    return pl.pallas_call(
        functools.partial(_fps_body, npoint),
        out_shape=out_shape,
        scratch_shapes=[pltpu.VMEM((Bb, Nn), jnp.float32)],
    )(px, py, pz)


def _ballq_body(r2, S, G, GS, NW, sx, sy, sz, qx, qy, qz, pmat, out_ref):
    Qb = qx.shape[2]
    dx = qx[0, 0, :][:, None] - sx[0, 0, :][None, :]
    dy = qy[0, 0, :][:, None] - sy[0, 0, :][None, :]
    dz = qz[0, 0, :][:, None] - sz[0, 0, :][None, :]
    d2 = dx * dx + dy * dy + dz * dz
    mask = (d2 <= r2).astype(jnp.bfloat16)
    words = jnp.dot(mask, pmat[...], preferred_element_type=jnp.float32)
    cnt = words[:, :G]
    D = cnt
    k = 1
    while k < G:
        D = D + jnp.concatenate(
            [jnp.zeros((Qb, k), jnp.float32), D[:, : G - k]], axis=1)
        k *= 2
    Dm1i = (D - cnt).astype(jnp.int32)
    Di = D.astype(jnp.int32)
    T = Di[:, G - 1:G]
    wi = [words[:, (1 + w) * G:(2 + w) * G].astype(jnp.int32)
          for w in range(NW)]
    iota_g = lax.broadcasted_iota(jnp.int32, (Qb, G), 1)
    iota_s = lax.broadcasted_iota(jnp.int32, (Qb, S), 1)

    def seat(s, carry):
        gsel, rsel, wsel = carry
        le = (Di <= s).astype(jnp.int32)
        g = jnp.sum(le, axis=1, keepdims=True)
        oh = iota_g == g
        cg = jnp.sum(jnp.where(oh, Dm1i, 0), axis=1, keepdims=True)
        r = s - cg
        ws = [jnp.sum(jnp.where(oh, w, 0), axis=1, keepdims=True) for w in wi]
        sm = iota_s == s
        gsel = jnp.where(sm, g, gsel)
        rsel = jnp.where(sm, r, rsel)
        wsel = [jnp.where(sm, w, ww) for w, ww in zip(ws, wsel)]
        return gsel, rsel, wsel

    z = jnp.zeros((Qb, S), jnp.int32)
    gsel, rsel, wsel = lax.fori_loop(
        0, S, seat, (z, z, [z for _ in range(NW)]))

    cum = jnp.zeros((Qb, S), jnp.int32)
    pos = jnp.zeros((Qb, S), jnp.int32)
    for w in range(NW):
        for p in range(16):
            bit = (wsel[w] >> p) & 1
            hit = (bit == 1) & (cum == rsel)
            pos = jnp.where(hit, w * 16 + p, pos)
            cum = cum + bit
    idx = gsel * GS + pos
    first = idx[:, 0:1]
    out_ref[0] = jnp.where(iota_s < T, idx, first)


def _ball_query(sx, sy, sz, qx, qy, qz, r, S, GS=64, Qb=128):
    Bb, Ns = sx.shape
    Q = qx.shape[1]
    G = Ns // GS
    NW = GS // 16
    pm = np.zeros((Ns, (1 + NW) * G), np.float32)
    for i in range(Ns):
        g = i // GS
        j = i % GS
        pm[i, g] = 1.0
        pm[i, (1 + j // 16) * G + g] = float(1 << (j % 16))
    pmat = jnp.asarray(pm, jnp.bfloat16)
    grid = (Bb, Q // Qb)
    src_spec = pl.BlockSpec((1, 1, Ns), lambda b, q: (b, 0, 0))
    q_spec = pl.BlockSpec((1, 1, Qb), lambda b, q: (b, 0, q))
    return pl.pallas_call(
        functools.partial(_ballq_body, r * r, S, G, GS, NW),
        grid=grid,
        in_specs=[src_spec] * 3 + [q_spec] * 3
        + [pl.BlockSpec((Ns, (1 + NW) * G), lambda b, q: (0, 0))],
        out_specs=pl.BlockSpec((1, Qb, S), lambda b, q: (b, q, 0)),
        out_shape=jax.ShapeDtypeStruct((Bb, Q, S), jnp.int32),
    )(sx.reshape(Bb, 1, Ns), sy.reshape(Bb, 1, Ns), sz.reshape(Bb, 1, Ns),
      qx.reshape(Bb, 1, Q), qy.reshape(Bb, 1, Q), qz.reshape(Bb, 1, Q), pmat)


def _bn_relu(z, m_ref, v_ref, g_ref, bt_ref):
    y = ((z - m_ref[0:1, :]) / jnp.sqrt(v_ref[0:1, :] + EPS) * g_ref[0:1, :]
         + bt_ref[0:1, :])
    return jnp.maximum(y, 0.0)


def _row_spec(c, br):
    return pl.BlockSpec((br, c), lambda i: (i, 0))


def _full2(a):
    return pl.BlockSpec(a.shape, lambda *_: (0, 0))


def _crow(c):
    return pl.BlockSpec((1, c), lambda *_: (0, 0))


def _start_body(x_ref, w_ref, b_ref, z_ref):
    z_ref[...] = jnp.dot(
        x_ref[...].astype(jnp.bfloat16), w_ref[...].astype(jnp.bfloat16),
        preferred_element_type=jnp.float32) + b_ref[0:1, :]


def _mlp_start(x, w, b, br=2048):
    R, ci = x.shape
    co = w.shape[1]
    return pl.pallas_call(
        _start_body,
        grid=(R // br,),
        in_specs=[_row_spec(ci, br), _full2(w), _crow(co)],
        out_specs=_row_spec(co, br),
        out_shape=jax.ShapeDtypeStruct((R, co), jnp.float32),
    )(x, w, b[None, :])


def _link_body(z_ref, m_ref, v_ref, g_ref, bt_ref, w_ref, b_ref, z2_ref):
    y = _bn_relu(z_ref[...], m_ref, v_ref, g_ref, bt_ref)
    z2_ref[...] = jnp.dot(
        y.astype(jnp.bfloat16), w_ref[...].astype(jnp.bfloat16),
        preferred_element_type=jnp.float32) + b_ref[0:1, :]


def _mlp_link(z, m, v, g, bt, w, b, br=2048):
    R, ci = z.shape
    co = w.shape[1]
    return pl.pallas_call(
        _link_body,
        grid=(R // br,),
        in_specs=[_row_spec(ci, br), _crow(ci), _crow(ci), _crow(ci),
                  _crow(ci), _full2(w), _crow(co)],
        out_specs=_row_spec(co, br),
        out_shape=jax.ShapeDtypeStruct((R, co), jnp.float32),
    )(z, m, v, g[None, :], bt[None, :], w, b[None, :])


def _finish_body(z_ref, m_ref, v_ref, g_ref, bt_ref, y_ref):
    y_ref[...] = _bn_relu(z_ref[...], m_ref, v_ref, g_ref, bt_ref)


def _mlp_finish(z, m, v, g, bt, br=2048):
    R, ci = z.shape
    return pl.pallas_call(
        _finish_body,
        grid=(R // br,),
        in_specs=[_row_spec(ci, br), _crow(ci), _crow(ci), _crow(ci),
                  _crow(ci)],
        out_specs=_row_spec(ci, br),
        out_shape=jax.ShapeDtypeStruct((R, ci), jnp.float32),
    )(z, m, v, g[None, :], bt[None, :])


def _finish_max_body(K, z_ref, m_ref, v_ref, g_ref, bt_ref, y_ref):
    y = _bn_relu(z_ref[...], m_ref, v_ref, g_ref, bt_ref)
    rb, c = z_ref.shape
    y_ref[...] = jnp.max(y.reshape(rb // K, K, c), axis=1)


def _mlp_finish_max(z, m, v, g, bt, K, qb=128):
    R, ci = z.shape
    br = qb * K
    return pl.pallas_call(
        functools.partial(_finish_max_body, K),
        grid=(R // br,),
        in_specs=[_row_spec(ci, br), _crow(ci), _crow(ci), _crow(ci),
                  _crow(ci)],
        out_specs=_row_spec(ci, qb),
        out_shape=jax.ShapeDtypeStruct((R // K, ci), jnp.float32),
    )(z, m, v, g[None, :], bt[None, :])


def _start_max_body(Nn, x_ref, w_ref, b_ref, z_ref):
    R, c = x_ref.shape
    mx = jnp.max(x_ref[...].reshape(R // Nn, Nn, c), axis=1)
    z_ref[...] = jnp.dot(
        mx.astype(jnp.bfloat16), w_ref[...].astype(jnp.bfloat16),
        preferred_element_type=jnp.float32) + b_ref[0:1, :]


def _mlp_start_max(x, Nn, w, b):
    R, ci = x.shape
    co = w.shape[1]
    return pl.pallas_call(
        functools.partial(_start_max_body, Nn),
        out_shape=jax.ShapeDtypeStruct((R // Nn, co), jnp.float32),
    )(x, w, b[None, :])


def _sa_l1_body(Sn, xw, plo, cx, cy, cz, w_ref, b_ref, rows_ref, z_ref):
    rows = rows_ref[...]
    cb = jnp.concatenate(
        [cx[0, 0, :][:, None], cy[0, 0, :][:, None], cz[0, 0, :][:, None]],
        axis=1)
    qb = cb.shape[0]
    cbr = jnp.broadcast_to(cb[:, None, :], (qb, Sn, 3)).reshape(qb * Sn, 3)
    feat = jnp.concatenate(
        [rows[:, :xw], rows[:, plo:plo + 3] - cbr], axis=1)
    z_ref[...] = jnp.dot(
        feat.astype(jnp.bfloat16), w_ref[...].astype(jnp.bfloat16),
        preferred_element_type=jnp.float32) + b_ref[0:1, :]


def _sa_l1(rows, cx, cy, cz, w, b, Q, Sn, xw, plo, qb=128):
    R, dpad = rows.shape
    c = w.shape[1]
    Bb = cx.shape[0]
    nq = Q // qb
    cspec = pl.BlockSpec((1, 1, qb), lambda b_, q: (b_, 0, q))
    return pl.pallas_call(
        functools.partial(_sa_l1_body, Sn, xw, plo),
        grid=(Bb, nq),
        in_specs=[cspec, cspec, cspec, _full2(w),
                  pl.BlockSpec((1, c), lambda b_, q: (0, 0)),
                  pl.BlockSpec((qb * Sn, dpad),
                               lambda b_, q: (b_ * nq + q, 0))],
        out_specs=pl.BlockSpec((qb * Sn, c), lambda b_, q: (b_ * nq + q, 0)),
        out_shape=jax.ShapeDtypeStruct((R, c), jnp.float32),
    )(cx.reshape(Bb, 1, Q), cy.reshape(Bb, 1, Q), cz.reshape(Bb, 1, Q),
      w, b[None, :], rows)


def _interp_body(k, sx, sy, sz, tx, ty, tz, x_ref, o_ref):
    Tb = tx.shape[2]
    Ns = sx.shape[2]
    dx = tx[0, 0, :][:, None] - sx[0, 0, :][None, :]
    dy = ty[0, 0, :][:, None] - sy[0, 0, :][None, :]
    dz = tz[0, 0, :][:, None] - sz[0, 0, :][None, :]
    d2 = dx * dx + dy * dy + dz * dz
    iota = lax.broadcasted_iota(jnp.int32, (Tb, Ns), 1)
    rw = jnp.zeros((Tb, Ns), jnp.float32)
    denom = jnp.zeros((Tb, 1), jnp.float32)
    for _ in range(k):
        mn = jnp.min(d2, axis=1, keepdims=True)
        am = jnp.argmin(d2, axis=1).astype(jnp.int32)[:, None]
        w = 1.0 / jnp.maximum(mn, 1e-16)
        oh = iota == am
        rw = rw + jnp.where(oh, w, 0.0)
        denom = denom + w
        d2 = jnp.where(oh, jnp.float32(1e30), d2)
    y = jnp.dot(rw, x_ref[0], preferred_element_type=jnp.float32,
                precision=lax.Precision.HIGHEST)
    o_ref[0] = y / denom


def _knn_interp(sx, sy, sz, tx, ty, tz, xsrc, tb=256):
    Bb, Ns = sx.shape
    Nt = tx.shape[1]
    C = xsrc.shape[2]
    sspec = pl.BlockSpec((1, 1, Ns), lambda b_, t: (b_, 0, 0))
    tspec = pl.BlockSpec((1, 1, tb), lambda b_, t: (b_, 0, t))
    return pl.pallas_call(
        functools.partial(_interp_body, KFP),
        grid=(Bb, Nt // tb),
        in_specs=[sspec, sspec, sspec, tspec, tspec, tspec,
                  pl.BlockSpec((1, Ns, C), lambda b_, t: (b_, 0, 0))],
        out_specs=pl.BlockSpec((1, tb, C), lambda b_, t: (b_, t, 0)),
        out_shape=jax.ShapeDtypeStruct((Bb, Nt, C), jnp.float32),
    )(sx.reshape(Bb, 1, Ns), sy.reshape(Bb, 1, Ns), sz.reshape(Bb, 1, Ns),
      tx.reshape(Bb, 1, Nt), ty.reshape(Bb, 1, Nt), tz.reshape(Bb, 1, Nt),
      xsrc)


def _sc_gather(table, idx):
    V, D = table.shape
    M = idx.shape[0]
    info = plsc.get_sparse_core_info()
    nw = info.num_cores * info.num_subcores
    per_w = M // nw
    assert per_w * nw == M
    ch = 128
    nch = per_w // ch
    assert nch * ch == per_w
    mesh = plsc.VectorSubcoreMesh(core_axis_name="c", subcore_axis_name="s")

    @functools.partial(
        pl.kernel, mesh=mesh,
        out_type=jax.ShapeDtypeStruct((M, D), jnp.float32),
        compiler_params=pltpu.CompilerParams(use_tc_tiling_on_sc=False),
        scratch_types=[pltpu.VMEM((ch,), jnp.int32),
                       pltpu.VMEM((ch, D), jnp.float32),
                       pltpu.SemaphoreType.DMA])
    def k(table_hbm, idx_hbm, out_hbm, idx_v, rows_v, sem):
        wid = lax.axis_index("s") * info.num_cores + lax.axis_index("c")
        base = wid * per_w

        def chunk(c_, carry):
            off = base + c_ * ch
            pltpu.sync_copy(idx_hbm.at[pl.ds(off, ch)], idx_v)
            pltpu.async_copy(table_hbm.at[idx_v], rows_v, sem).wait()
            pltpu.sync_copy(rows_v, out_hbm.at[pl.ds(off, ch)])
            return carry

        lax.fori_loop(0, nch, chunk, 0)

    return k(table, idx)


def _bn_relu_x(x, g, b):
    xs = x.reshape(-1, x.shape[-1])
    mean = xs.mean(axis=0)
    var = xs.var(axis=0)
    y = (x - mean) / jnp.sqrt(var + EPS) * g + b
    return jnp.maximum(y, 0.0)


def _apply_mlp_x(x, layers):
    for (Wl, bb, gm, bt) in layers:
        x = _bn_relu_x(x @ Wl + bb, gm, bt)
    return x


def kernel(pts, params):
    ptsx, ptsy, ptsz = pts[:, :, 0], pts[:, :, 1], pts[:, :, 2]

    _, p1x, p1y, p1z = _fps(ptsx, ptsy, ptsz, N1)
    pos1 = jnp.stack([p1x, p1y, p1z], axis=-1)
    g1 = _ball_query(ptsx, ptsy, ptsz, p1x, p1y, p1z, R1, MAXN1)
    tab1 = jnp.concatenate(
        [pts.reshape(B * N, 3),
         jnp.zeros((B * N, 13), jnp.float32)], axis=1)
    offs = (jnp.arange(B, dtype=jnp.int32) * N)[:, None]
    gidx1 = (g1.reshape(B, -1) + offs).reshape(-1)
    rows1 = _sc_gather(tab1, gidx1)
    pj = rows1[:, :3].reshape(B, N1, MAXN1, 3)
    feat = jnp.concatenate([pj, pj - pos1[:, :, None, :]], axis=-1)
    h = _apply_mlp_x(feat, params['sa1_local'])
    x1 = _apply_mlp_x(jnp.max(h, axis=2), params['sa1_global'])

    _, p2x, p2y, p2z = _fps(p1x, p1y, p1z, N2)
    pos2 = jnp.stack([p2x, p2y, p2z], axis=-1)
    g2 = _ball_query(p1x, p1y, p1z, p2x, p2y, p2z, R2, MAXN2)
    tab2 = jnp.concatenate(
        [x1.reshape(B * N1, 256), pos1.reshape(B * N1, 3),
         jnp.zeros((B * N1, 13), jnp.float32)], axis=1)
    offs1 = (jnp.arange(B, dtype=jnp.int32) * N1)[:, None]
    gidx2 = (g2.reshape(B, -1) + offs1).reshape(-1)
    rows2 = _sc_gather(tab2, gidx2)
    xj = rows2[:, :256].reshape(B, N2, MAXN2, 256)
    pj2 = rows2[:, 256:259].reshape(B, N2, MAXN2, 3)
    feat2 = jnp.concatenate([xj, pj2 - pos2[:, :, None, :]], axis=-1)
    hh = _apply_mlp_x(feat2, params['sa2_local'])
    x2 = _apply_mlp_x(jnp.max(hh, axis=2), params['sa2_global'])

    gg = _apply_mlp_x(jnp.max(x2, axis=1), params['glob'])

    x1_up = _knn_interp(p2x, p2y, p2z, p1x, p1y, p1z, x2, tb=256)
    x1_fp = _apply_mlp_x(jnp.concatenate([x1_up, x1], axis=-1),
                         params['fp1'])

    x0_up = _knn_interp(p1x, p1y, p1z, ptsx, ptsy, ptsz, x1_fp, tb=256)
    F = _apply_mlp_x(jnp.concatenate([x0_up, pts], axis=-1), params['fp0'])
    return (F, gg)
